# Initial kernel scaffold; baseline (speedup 1.0000x reference)
import jax
import jax.numpy as jnp
from jax.experimental import pallas as pl


def kernel(img, feat0, feat1, sample_patch, sample_locs, W0a, b0a, W0b, b0b, W1a, b1a, W1b, b1b):
    raise NotImplementedError("write your pallas kernel here")


# speedup vs baseline: 8.9045x; 8.9045x over previous
import jax
import jax.numpy as jnp
from jax import lax
from jax.experimental import pallas as pl
from jax.experimental.pallas import tpu as pltpu

_NC = 256
_NSEL = 256
_COUNT = 1600
_PAD = 1664
_NB = _PAD // 128
_PH = _PW = 32
_HALF = 16
_BIG = 3.0e38


def _body(locs_s, img_ref, sp_ref, lr_ref, lc_ref, f0_ref, f1_ref,
          w0a_ref, b0a_ref, w0b_ref, b0b_ref,
          w1a_ref, b1a_ref, w1b_ref, b1b_ref,
          out0_ref, out1_ref, ids_ref,
          scores_ref, rank_ref):
    sp = sp_ref[...]
    lane = lax.broadcasted_iota(jnp.int32, (128,), 0)

    def score_block(blk, _):
        b0 = pl.multiple_of(blk * 128, 128)

        def score_one(k, acc):
            i = blk * 128 + k
            r = locs_s[i, 0]
            c = locs_s[i, 1]
            c0 = c - _HALF
            base = pl.multiple_of(jnp.minimum(c0 // 128, 2) * 128, 128)
            co = c0 - base
            slab = img_ref[pl.ds(r - _HALF, _PH), :, pl.ds(base, 256)]
            rot = pltpu.roll(slab, (256 - co) % 256, axis=2)
            d = 1.0 - jnp.square(rot[:, :, :_PW] - sp)
            rows = jnp.sum(d, axis=2)
            hi = lax.round(rows * 128.0,
                           lax.RoundingMethod.TO_NEAREST_EVEN) * 0.0078125
            val = jnp.sum(hi) + jnp.sum(rows - hi)
            return jnp.where(lane == k, val, acc)

        acc = lax.fori_loop(0, 128, score_one, jnp.zeros((128,), jnp.float32))
        acc = jnp.where(b0 + lane < _COUNT, acc, _BIG)
        scores_ref[pl.ds(b0, 128)] = acc
        return 0

    lax.fori_loop(0, _NB, score_block, 0)

    sall = scores_ref[...]
    jj = lax.broadcasted_iota(jnp.int32, (16, _PAD), 1)

    def rank_block(b, _):
        b0 = pl.multiple_of(b * 128, 128)
        sblk = scores_ref[pl.ds(b0, 128)]
        parts = []
        for sub in range(8):
            sr = sblk[sub * 16:(sub + 1) * 16][:, None]
            ri = (b * 128 + sub * 16
                  + lax.broadcasted_iota(jnp.int32, (16, 1), 0))
            less = (sall[None, :] < sr) | ((sall[None, :] == sr) & (jj < ri))
            parts.append(jnp.sum(less.astype(jnp.float32), axis=1))
        rank_ref[pl.ds(b0, 128)] = jnp.concatenate(parts)
        return 0

    lax.fori_loop(0, _NB, rank_block, 0)

    kio = lax.broadcasted_iota(jnp.int32, (_NSEL, 1), 0).astype(jnp.float32)

    def sel_chunk(ch, carry):
        ar, ac = carry
        c0 = pl.multiple_of(ch * 128, 128)
        rk = rank_ref[pl.ds(c0, 128)]
        eq = (rk[None, :] == kio).astype(jnp.int32)
        lr = lr_ref[pl.ds(c0, 128)]
        lc = lc_ref[pl.ds(c0, 128)]
        ar = ar + jnp.sum(eq * lr[None, :], axis=1)
        ac = ac + jnp.sum(eq * lc[None, :], axis=1)
        return ar, ac

    zero = jnp.zeros((_NSEL,), jnp.int32)
    ar, ac = lax.fori_loop(0, _NB, sel_chunk, (zero, zero))
    ids_ref[:, 0:1] = ar[:, None]
    ids_ref[:, 1:2] = ac[:, None]

    cell0 = (ar // 8) * 64 + (ac // 8)
    cell1 = (ar // 16) * 32 + (ac // 16)

    hw0 = lax.broadcasted_iota(jnp.int32, (_NSEL, 4096), 1)
    oh0 = (cell0[:, None] == hw0).astype(jnp.float32)
    hw1 = lax.broadcasted_iota(jnp.int32, (_NSEL, 1024), 1)
    oh1 = (cell1[:, None] == hw1).astype(jnp.float32)

    dn = (((1,), (1,)), ((), ()))
    g0 = lax.dot_general(oh0, f0_ref[...], dn,
                         preferred_element_type=jnp.float32)
    g1 = lax.dot_general(oh1, f1_ref[...], dn,
                         preferred_element_type=jnp.float32)

    hp = jax.lax.Precision.HIGHEST
    h0 = jnp.maximum(
        jnp.dot(g0, w0a_ref[...], precision=hp,
                preferred_element_type=jnp.float32) + b0a_ref[...][None, :],
        0.0)
    y0 = jnp.dot(h0, w0b_ref[...], precision=hp,
                 preferred_element_type=jnp.float32) + b0b_ref[...][None, :]
    n0 = jnp.maximum(jnp.sqrt(jnp.sum(y0 * y0, axis=1, keepdims=True)),
                     1e-12)
    out0_ref[...] = y0 / n0

    h1 = jnp.maximum(
        jnp.dot(g1, w1a_ref[...], precision=hp,
                preferred_element_type=jnp.float32) + b1a_ref[...][None, :],
        0.0)
    y1 = jnp.dot(h1, w1b_ref[...], precision=hp,
                 preferred_element_type=jnp.float32) + b1b_ref[...][None, :]
    n1 = jnp.maximum(jnp.sqrt(jnp.sum(y1 * y1, axis=1, keepdims=True)),
                     1e-12)
    out1_ref[...] = y1 / n1


def kernel(img, feat0, feat1, sample_patch, sample_locs,
           W0a, b0a, W0b, b0b, W1a, b1a, W1b, b1b):
    img3 = img[0].transpose(1, 0, 2)
    sp = sample_patch[0].transpose(1, 0, 2)
    f0 = feat0[0].reshape(256, 64 * 64)
    f1 = feat1[0].reshape(512, 32 * 32)
    locs = sample_locs.astype(jnp.int32)
    locs_pad = jnp.pad(locs, ((0, _PAD - _COUNT), (0, 0)),
                       constant_values=_HALF)
    lr = locs_pad[:, 0]
    lc = locs_pad[:, 1]

    out0, out1, ids = pl.pallas_call(
        _body,
        grid_spec=pltpu.PrefetchScalarGridSpec(
            num_scalar_prefetch=1,
            in_specs=[pl.BlockSpec(memory_space=pltpu.VMEM)] * 14,
            out_specs=[pl.BlockSpec(memory_space=pltpu.VMEM)] * 3,
            scratch_shapes=[
                pltpu.VMEM((_PAD,), jnp.float32),
                pltpu.VMEM((_PAD,), jnp.float32),
            ],
        ),
        out_shape=[
            jax.ShapeDtypeStruct((_NSEL, _NC), jnp.float32),
            jax.ShapeDtypeStruct((_NSEL, _NC), jnp.float32),
            jax.ShapeDtypeStruct((_NSEL, 2), jnp.int32),
        ],
    )(locs_pad, img3, sp, lr, lc, f0, f1,
      W0a, b0a, W0b, b0b, W1a, b1a, W1b, b1b)
    return out0, out1, ids


# speedup vs baseline: 34.5097x; 3.8755x over previous
import functools

import jax
import jax.numpy as jnp
from jax import lax
from jax.experimental import pallas as pl
from jax.experimental.pallas import tpu as pltpu
from jax.experimental.pallas import tpu_sc as plsc

_NC = 256
_NSEL = 256
_COUNT = 1600
_PAD = 1664
_NB = _PAD // 128
_PH = _PW = 32
_HALF = 16
_BIG = 3.0e38
_NW = 32
_CPW = _COUNT // _NW
_Q = 0.0078125


def _sc_scores_body(img_hbm, locs_hbm, sp_hbm, out_hbm,
                    locs_v, sp_v, patch_v, scores_v):
    wid = lax.axis_index("s") * 2 + lax.axis_index("c")
    pltpu.sync_copy(locs_hbm.at[wid], locs_v)
    pltpu.sync_copy(sp_hbm, sp_v)

    def one_candidate(j, carry):
        lv = locs_v[pl.ds(2 * j, 16)]
        r = lv[0]
        c = lv[1]
        c0 = c - _HALF
        cbase = pl.multiple_of((c0 // 8) * 8, 8)
        o = c0 - cbase
        for ch in range(3):
            pltpu.sync_copy(
                img_hbm.at[ch, pl.ds(r - _HALF, _PH), pl.ds(cbase, 40)],
                patch_v.at[ch])

        def row_block(b, acc):
            def one_row(t, sub):
                ch2 = t // _PH
                row = t % _PH
                x0 = patch_v[ch2, row, pl.ds(o, 16)]
                x1 = patch_v[ch2, row, pl.ds(o + 16, 16)]
                s0 = sp_v[ch2, row, pl.ds(0, 16)]
                s1 = sp_v[ch2, row, pl.ds(16, 16)]
                d0 = x0 - s0
                d1 = x1 - s1
                return sub + (d0 * d0 + d1 * d1)

            sub = lax.fori_loop(b * 16, (b + 1) * 16, one_row,
                                jnp.zeros((16,), jnp.float32))
            return acc + sub

        acc = lax.fori_loop(0, 6, row_block, jnp.zeros((16,), jnp.float32))
        hi = (acc * 128.0).astype(jnp.int32)
        lo = acc - hi.astype(jnp.float32) * _Q
        ssd = jnp.sum(hi).astype(jnp.float32) * _Q + jnp.sum(lo)
        score = 3072.0 - ssd
        m = j // 16
        ln = j % 16
        lane16 = lax.broadcasted_iota(jnp.int32, (16,), 0)
        return tuple(
            jnp.where((m == mi) & (lane16 == ln), score, sm)
            for mi, sm in enumerate(carry))

    init = (jnp.zeros((16,), jnp.float32),) * 4
    svecs = lax.fori_loop(0, _CPW, one_candidate, init)
    for mi, sm in enumerate(svecs):
        scores_v[pl.ds(mi * 16, 16)] = sm
    pltpu.sync_copy(scores_v, out_hbm.at[wid])


_sc_scores = functools.partial(
    pl.kernel,
    mesh=plsc.VectorSubcoreMesh(core_axis_name="c", subcore_axis_name="s",
                                num_cores=2),
    compiler_params=pltpu.CompilerParams(use_tc_tiling_on_sc=False,
                                         needs_layout_passes=False),
    out_type=jax.ShapeDtypeStruct((_NW, 64), jnp.float32),
    scratch_types=[
        pltpu.VMEM((120,), jnp.int32),
        pltpu.VMEM((3, _PH, _PW), jnp.float32),
        pltpu.VMEM((3, _PH, 40), jnp.float32),
        pltpu.VMEM((64,), jnp.float32),
    ],
)(_sc_scores_body)


def _tc_body(scores_ref, lr_ref, lc_ref, f0_ref, f1_ref,
             w0a_ref, b0a_ref, w0b_ref, b0b_ref,
             w1a_ref, b1a_ref, w1b_ref, b1b_ref,
             out0_ref, out1_ref, ids_ref,
             rank_ref):
    sall = scores_ref[...]
    jj = lax.broadcasted_iota(jnp.int32, (16, _PAD), 1)

    def rank_block(b, _):
        b0 = pl.multiple_of(b * 128, 128)
        sblk = scores_ref[pl.ds(b0, 128)]
        parts = []
        for sub in range(8):
            sr = sblk[sub * 16:(sub + 1) * 16][:, None]
            ri = (b * 128 + sub * 16
                  + lax.broadcasted_iota(jnp.int32, (16, 1), 0))
            less = (sall[None, :] < sr) | ((sall[None, :] == sr) & (jj < ri))
            parts.append(jnp.sum(less.astype(jnp.float32), axis=1))
        rank_ref[pl.ds(b0, 128)] = jnp.concatenate(parts)
        return 0

    lax.fori_loop(0, _NB, rank_block, 0)

    kio = lax.broadcasted_iota(jnp.int32, (_NSEL, 1), 0).astype(jnp.float32)

    def sel_chunk(ch, carry):
        ar, ac = carry
        c0 = pl.multiple_of(ch * 128, 128)
        rk = rank_ref[pl.ds(c0, 128)]
        eq = (rk[None, :] == kio).astype(jnp.int32)
        lr = lr_ref[pl.ds(c0, 128)]
        lc = lc_ref[pl.ds(c0, 128)]
        ar = ar + jnp.sum(eq * lr[None, :], axis=1)
        ac = ac + jnp.sum(eq * lc[None, :], axis=1)
        return ar, ac

    zero = jnp.zeros((_NSEL,), jnp.int32)
    ar, ac = lax.fori_loop(0, _NB, sel_chunk, (zero, zero))
    ids_ref[:, 0:1] = ar[:, None]
    ids_ref[:, 1:2] = ac[:, None]

    cell0 = (ar // 8) * 64 + (ac // 8)
    cell1 = (ar // 16) * 32 + (ac // 16)

    hw0 = lax.broadcasted_iota(jnp.int32, (_NSEL, 4096), 1)
    oh0 = (cell0[:, None] == hw0).astype(jnp.float32)
    hw1 = lax.broadcasted_iota(jnp.int32, (_NSEL, 1024), 1)
    oh1 = (cell1[:, None] == hw1).astype(jnp.float32)

    dn = (((1,), (1,)), ((), ()))
    g0 = lax.dot_general(oh0, f0_ref[...], dn,
                         preferred_element_type=jnp.float32)
    g1 = lax.dot_general(oh1, f1_ref[...], dn,
                         preferred_element_type=jnp.float32)

    hp = jax.lax.Precision.HIGHEST
    h0 = jnp.maximum(
        jnp.dot(g0, w0a_ref[...], precision=hp,
                preferred_element_type=jnp.float32) + b0a_ref[...][None, :],
        0.0)
    y0 = jnp.dot(h0, w0b_ref[...], precision=hp,
                 preferred_element_type=jnp.float32) + b0b_ref[...][None, :]
    n0 = jnp.maximum(jnp.sqrt(jnp.sum(y0 * y0, axis=1, keepdims=True)),
                     1e-12)
    out0_ref[...] = y0 / n0

    h1 = jnp.maximum(
        jnp.dot(g1, w1a_ref[...], precision=hp,
                preferred_element_type=jnp.float32) + b1a_ref[...][None, :],
        0.0)
    y1 = jnp.dot(h1, w1b_ref[...], precision=hp,
                 preferred_element_type=jnp.float32) + b1b_ref[...][None, :]
    n1 = jnp.maximum(jnp.sqrt(jnp.sum(y1 * y1, axis=1, keepdims=True)),
                     1e-12)
    out1_ref[...] = y1 / n1


def kernel(img, feat0, feat1, sample_patch, sample_locs,
           W0a, b0a, W0b, b0b, W1a, b1a, W1b, b1b):
    img3 = img[0]
    sp = sample_patch[0]
    f0 = feat0[0].reshape(256, 64 * 64)
    f1 = feat1[0].reshape(512, 32 * 32)
    locs = sample_locs.astype(jnp.int32)
    locs_w = jnp.pad(locs.reshape(_NW, 2 * _CPW),
                     ((0, 0), (0, 20)))

    scores_w = _sc_scores(img3, locs_w, sp)
    scores = scores_w[:, :_CPW].reshape(_COUNT)
    scores_pad = jnp.concatenate(
        [scores, jnp.full((_PAD - _COUNT,), _BIG, jnp.float32)])

    lr = jnp.pad(locs[:, 0], (0, _PAD - _COUNT))
    lc = jnp.pad(locs[:, 1], (0, _PAD - _COUNT))

    out0, out1, ids = pl.pallas_call(
        _tc_body,
        in_specs=[pl.BlockSpec(memory_space=pltpu.VMEM)] * 13,
        out_specs=[pl.BlockSpec(memory_space=pltpu.VMEM)] * 3,
        scratch_shapes=[
            pltpu.VMEM((_PAD,), jnp.float32),
        ],
        out_shape=[
            jax.ShapeDtypeStruct((_NSEL, _NC), jnp.float32),
            jax.ShapeDtypeStruct((_NSEL, _NC), jnp.float32),
            jax.ShapeDtypeStruct((_NSEL, 2), jnp.int32),
        ],
    )(scores_pad, lr, lc, f0, f1,
      W0a, b0a, W0b, b0b, W1a, b1a, W1b, b1b)
    return out0, out1, ids


# speedup vs baseline: 70.2133x; 2.0346x over previous
import functools

import jax
import jax.numpy as jnp
from jax import lax
from jax.experimental import pallas as pl
from jax.experimental.pallas import tpu as pltpu
from jax.experimental.pallas import tpu_sc as plsc

_NC = 256
_NSEL = 256
_COUNT = 1600
_PAD = 1664
_NB = _PAD // 128
_PH = _PW = 32
_HALF = 16
_BIG = 3.0e38
_NW = 32
_CPW = _COUNT // _NW
_Q = 0.0078125


def _sc_scores_body(img_hbm, locs_hbm, sp_hbm, out_hbm,
                    locs_v, sp_v, pa_v, pb_v, scores_v, sema, semb):
    wid = lax.axis_index("s") * 2 + lax.axis_index("c")
    pltpu.sync_copy(locs_hbm.at[wid], locs_v)
    pltpu.sync_copy(sp_hbm, sp_v)
    lane16 = lax.broadcasted_iota(jnp.int32, (16,), 0)

    def rcbase(j):
        lv = locs_v[pl.ds(2 * j, 16)]
        r0 = lv[0] - _HALF
        c0 = lv[1] - _HALF
        cb = pl.multiple_of((c0 // 8) * 8, 8)
        return r0, cb, c0 - cb

    def issue(j, buf, sem):
        r0, cb, _ = rcbase(j)
        for ch in range(3):
            pltpu.make_async_copy(
                img_hbm.at[ch, pl.ds(r0, _PH), pl.ds(cb, 40)],
                buf.at[ch], sem).start()

    def wait(buf, sem):
        for ch in range(3):
            pltpu.make_async_copy(
                img_hbm.at[ch, pl.ds(0, _PH), pl.ds(0, 40)],
                buf.at[ch], sem).wait()

    def compute(j, buf):
        _, _, o = rcbase(j)
        subs = []
        for ch in range(3):
            for half in range(2):
                def row_f(row, a, ch=ch):
                    x0 = buf[ch, row, pl.ds(o, 16)]
                    x1 = buf[ch, row, pl.ds(o + 16, 16)]
                    s0 = sp_v[ch, row, pl.ds(0, 16)]
                    s1 = sp_v[ch, row, pl.ds(16, 16)]
                    d0 = x0 - s0
                    d1 = x1 - s1
                    return a + (d0 * d0 + d1 * d1)

                subs.append(lax.fori_loop(half * 16, (half + 1) * 16, row_f,
                                          jnp.zeros((16,), jnp.float32)))
        acc = ((subs[0] + subs[1]) + (subs[2] + subs[3])) + (subs[4] + subs[5])
        hi = (acc * 128.0).astype(jnp.int32)
        lo = acc - hi.astype(jnp.float32) * _Q
        ssd = jnp.sum(hi).astype(jnp.float32) * _Q + jnp.sum(lo)
        return 3072.0 - ssd

    def place(j, score, svecs):
        m = j // 16
        ln = j % 16
        return tuple(
            jnp.where((m == mi) & (lane16 == ln), score, sm)
            for mi, sm in enumerate(svecs))

    issue(0, pa_v, sema)

    def pair(m, svecs):
        issue(2 * m + 1, pb_v, semb)
        wait(pa_v, sema)
        svecs = place(2 * m, compute(2 * m, pa_v), svecs)

        @pl.when(m < _CPW // 2 - 1)
        def _():
            issue(2 * m + 2, pa_v, sema)

        wait(pb_v, semb)
        svecs = place(2 * m + 1, compute(2 * m + 1, pb_v), svecs)
        return svecs

    init = (jnp.zeros((16,), jnp.float32),) * 4
    svecs = lax.fori_loop(0, _CPW // 2, pair, init)
    for mi, sm in enumerate(svecs):
        scores_v[pl.ds(mi * 16, 16)] = sm
    pltpu.sync_copy(scores_v, out_hbm.at[wid])


_sc_scores = functools.partial(
    pl.kernel,
    mesh=plsc.VectorSubcoreMesh(core_axis_name="c", subcore_axis_name="s",
                                num_cores=2),
    compiler_params=pltpu.CompilerParams(use_tc_tiling_on_sc=False,
                                         needs_layout_passes=False),
    out_type=jax.ShapeDtypeStruct((_NW, 64), jnp.float32),
    scratch_types=[
        pltpu.VMEM((120,), jnp.int32),
        pltpu.VMEM((3, _PH, _PW), jnp.float32),
        pltpu.VMEM((3, _PH, 40), jnp.float32),
        pltpu.VMEM((3, _PH, 40), jnp.float32),
        pltpu.VMEM((64,), jnp.float32),
        pltpu.SemaphoreType.DMA,
        pltpu.SemaphoreType.DMA,
    ],
)(_sc_scores_body)


def _tc_body(scores_ref, lr_ref, lc_ref, f0_ref, f1_ref,
             w0a_ref, b0a_ref, w0b_ref, b0b_ref,
             w1a_ref, b1a_ref, w1b_ref, b1b_ref,
             out0_ref, out1_ref, ids_ref,
             rank_ref):
    sall = scores_ref[...]
    jj = lax.broadcasted_iota(jnp.int32, (16, _PAD), 1)

    def rank_block(b, _):
        b0 = pl.multiple_of(b * 128, 128)
        sblk = scores_ref[pl.ds(b0, 128)]
        parts = []
        for sub in range(8):
            sr = sblk[sub * 16:(sub + 1) * 16][:, None]
            ri = (b * 128 + sub * 16
                  + lax.broadcasted_iota(jnp.int32, (16, 1), 0))
            less = (sall[None, :] < sr) | ((sall[None, :] == sr) & (jj < ri))
            parts.append(jnp.sum(less.astype(jnp.float32), axis=1))
        rank_ref[pl.ds(b0, 128)] = jnp.concatenate(parts)
        return 0

    lax.fori_loop(0, _NB, rank_block, 0)

    kio = lax.broadcasted_iota(jnp.int32, (_NSEL, 1), 0).astype(jnp.float32)

    def sel_chunk(ch, carry):
        ar, ac = carry
        c0 = pl.multiple_of(ch * 128, 128)
        rk = rank_ref[pl.ds(c0, 128)]
        eq = (rk[None, :] == kio).astype(jnp.int32)
        lr = lr_ref[pl.ds(c0, 128)]
        lc = lc_ref[pl.ds(c0, 128)]
        ar = ar + jnp.sum(eq * lr[None, :], axis=1)
        ac = ac + jnp.sum(eq * lc[None, :], axis=1)
        return ar, ac

    zero = jnp.zeros((_NSEL,), jnp.int32)
    ar, ac = lax.fori_loop(0, _NB, sel_chunk, (zero, zero))
    ids_ref[:, 0:1] = ar[:, None]
    ids_ref[:, 1:2] = ac[:, None]

    cell0 = (ar // 8) * 64 + (ac // 8)
    cell1 = (ar // 16) * 32 + (ac // 16)

    hw0 = lax.broadcasted_iota(jnp.int32, (_NSEL, 4096), 1)
    oh0 = (cell0[:, None] == hw0).astype(jnp.float32)
    hw1 = lax.broadcasted_iota(jnp.int32, (_NSEL, 1024), 1)
    oh1 = (cell1[:, None] == hw1).astype(jnp.float32)

    dn = (((1,), (1,)), ((), ()))
    g0 = lax.dot_general(oh0, f0_ref[...], dn,
                         preferred_element_type=jnp.float32)
    g1 = lax.dot_general(oh1, f1_ref[...], dn,
                         preferred_element_type=jnp.float32)

    hp = jax.lax.Precision.HIGHEST
    h0 = jnp.maximum(
        jnp.dot(g0, w0a_ref[...], precision=hp,
                preferred_element_type=jnp.float32) + b0a_ref[...][None, :],
        0.0)
    y0 = jnp.dot(h0, w0b_ref[...], precision=hp,
                 preferred_element_type=jnp.float32) + b0b_ref[...][None, :]
    n0 = jnp.maximum(jnp.sqrt(jnp.sum(y0 * y0, axis=1, keepdims=True)),
                     1e-12)
    out0_ref[...] = y0 / n0

    h1 = jnp.maximum(
        jnp.dot(g1, w1a_ref[...], precision=hp,
                preferred_element_type=jnp.float32) + b1a_ref[...][None, :],
        0.0)
    y1 = jnp.dot(h1, w1b_ref[...], precision=hp,
                 preferred_element_type=jnp.float32) + b1b_ref[...][None, :]
    n1 = jnp.maximum(jnp.sqrt(jnp.sum(y1 * y1, axis=1, keepdims=True)),
                     1e-12)
    out1_ref[...] = y1 / n1


def kernel(img, feat0, feat1, sample_patch, sample_locs,
           W0a, b0a, W0b, b0b, W1a, b1a, W1b, b1b):
    img3 = img[0]
    sp = sample_patch[0]
    f0 = feat0[0].reshape(256, 64 * 64)
    f1 = feat1[0].reshape(512, 32 * 32)
    locs = sample_locs.astype(jnp.int32)
    locs_w = jnp.pad(locs.reshape(_NW, 2 * _CPW),
                     ((0, 0), (0, 20)))

    scores_w = _sc_scores(img3, locs_w, sp)
    scores = scores_w[:, :_CPW].reshape(_COUNT)
    scores_pad = jnp.concatenate(
        [scores, jnp.full((_PAD - _COUNT,), _BIG, jnp.float32)])

    lr = jnp.pad(locs[:, 0], (0, _PAD - _COUNT))
    lc = jnp.pad(locs[:, 1], (0, _PAD - _COUNT))

    out0, out1, ids = pl.pallas_call(
        _tc_body,
        in_specs=[pl.BlockSpec(memory_space=pltpu.VMEM)] * 13,
        out_specs=[pl.BlockSpec(memory_space=pltpu.VMEM)] * 3,
        scratch_shapes=[
            pltpu.VMEM((_PAD,), jnp.float32),
        ],
        out_shape=[
            jax.ShapeDtypeStruct((_NSEL, _NC), jnp.float32),
            jax.ShapeDtypeStruct((_NSEL, _NC), jnp.float32),
            jax.ShapeDtypeStruct((_NSEL, 2), jnp.int32),
        ],
    )(scores_pad, lr, lc, f0, f1,
      W0a, b0a, W0b, b0b, W1a, b1a, W1b, b1b)
    return out0, out1, ids


# speedup vs baseline: 72.6858x; 1.0352x over previous
import functools

import jax
import jax.numpy as jnp
from jax import lax
from jax.experimental import pallas as pl
from jax.experimental.pallas import tpu as pltpu
from jax.experimental.pallas import tpu_sc as plsc

_NC = 256
_NSEL = 256
_COUNT = 1600
_PAD = 1664
_NB = _PAD // 128
_PH = _PW = 32
_HALF = 16
_BIG = 3.0e38
_NW = 32
_CPW = _COUNT // _NW
_Q = 0.0078125


def _sc_scores_body(img_hbm, locs_hbm, sp_hbm, out_hbm,
                    locs_v, sp_v, pa_v, pb_v, scores_v, sema, semb):
    wid = lax.axis_index("s") * 2 + lax.axis_index("c")
    pltpu.sync_copy(locs_hbm.at[wid], locs_v)
    pltpu.sync_copy(sp_hbm, sp_v)
    lane16 = lax.broadcasted_iota(jnp.int32, (16,), 0)

    def rcbase(j):
        lv = locs_v[pl.ds(2 * j, 16)]
        r0 = lv[0] - _HALF
        c0 = lv[1] - _HALF
        cb = pl.multiple_of((c0 // 8) * 8, 8)
        return r0, cb, c0 - cb

    def issue(j, buf, sem):
        r0, cb, _ = rcbase(j)
        for ch in range(3):
            pltpu.make_async_copy(
                img_hbm.at[ch, pl.ds(r0, _PH), pl.ds(cb, 40)],
                buf.at[ch], sem).start()

    def wait(buf, sem):
        for ch in range(3):
            pltpu.make_async_copy(
                img_hbm.at[ch, pl.ds(0, _PH), pl.ds(0, 40)],
                buf.at[ch], sem).wait()

    def compute(j, buf):
        _, _, o = rcbase(j)
        subs = []
        for ch in range(3):
            for half in range(2):
                def row_f(t, a, ch=ch, base=half * 16):
                    acc2 = a
                    for dr in range(2):
                        row = base + 2 * t + dr
                        x0 = buf[ch, row, pl.ds(o, 16)]
                        x1 = buf[ch, row, pl.ds(o + 16, 16)]
                        s0 = sp_v[ch, row, pl.ds(0, 16)]
                        s1 = sp_v[ch, row, pl.ds(16, 16)]
                        d0 = x0 - s0
                        d1 = x1 - s1
                        acc2 = acc2 + (d0 * d0 + d1 * d1)
                    return acc2

                subs.append(lax.fori_loop(0, 8, row_f,
                                          jnp.zeros((16,), jnp.float32)))
        acc = ((subs[0] + subs[1]) + (subs[2] + subs[3])) + (subs[4] + subs[5])
        hi = (acc * 128.0).astype(jnp.int32)
        lo = acc - hi.astype(jnp.float32) * _Q
        ssd = jnp.sum(hi).astype(jnp.float32) * _Q + jnp.sum(lo)
        return 3072.0 - ssd

    def place(j, score, svecs):
        m = j // 16
        ln = j % 16
        return tuple(
            jnp.where((m == mi) & (lane16 == ln), score, sm)
            for mi, sm in enumerate(svecs))

    issue(0, pa_v, sema)

    def pair(m, svecs):
        issue(2 * m + 1, pb_v, semb)
        wait(pa_v, sema)
        svecs = place(2 * m, compute(2 * m, pa_v), svecs)

        @pl.when(m < _CPW // 2 - 1)
        def _():
            issue(2 * m + 2, pa_v, sema)

        wait(pb_v, semb)
        svecs = place(2 * m + 1, compute(2 * m + 1, pb_v), svecs)
        return svecs

    init = (jnp.zeros((16,), jnp.float32),) * 4
    svecs = lax.fori_loop(0, _CPW // 2, pair, init)
    for mi, sm in enumerate(svecs):
        scores_v[pl.ds(mi * 16, 16)] = sm
    pltpu.sync_copy(scores_v, out_hbm.at[wid])


_sc_scores = functools.partial(
    pl.kernel,
    mesh=plsc.VectorSubcoreMesh(core_axis_name="c", subcore_axis_name="s",
                                num_cores=2),
    compiler_params=pltpu.CompilerParams(use_tc_tiling_on_sc=False,
                                         needs_layout_passes=False),
    out_type=jax.ShapeDtypeStruct((_NW, 64), jnp.float32),
    scratch_types=[
        pltpu.VMEM((120,), jnp.int32),
        pltpu.VMEM((3, _PH, _PW), jnp.float32),
        pltpu.VMEM((3, _PH, 40), jnp.float32),
        pltpu.VMEM((3, _PH, 40), jnp.float32),
        pltpu.VMEM((64,), jnp.float32),
        pltpu.SemaphoreType.DMA,
        pltpu.SemaphoreType.DMA,
    ],
)(_sc_scores_body)


def _tc_body(scores_ref, lr_ref, lc_ref, f0_ref, f1_ref,
             w0a_ref, b0a_ref, w0b_ref, b0b_ref,
             w1a_ref, b1a_ref, w1b_ref, b1b_ref,
             out0_ref, out1_ref, ids_ref,
             rank_ref):
    sall = scores_ref[...]
    jj = lax.broadcasted_iota(jnp.int32, (16, _PAD), 1)

    def rank_block(b, _):
        b0 = pl.multiple_of(b * 128, 128)
        sblk = scores_ref[pl.ds(b0, 128)]
        parts = []
        for sub in range(8):
            sr = sblk[sub * 16:(sub + 1) * 16][:, None]
            ri = (b * 128 + sub * 16
                  + lax.broadcasted_iota(jnp.int32, (16, 1), 0))
            less = (sall[None, :] < sr) | ((sall[None, :] == sr) & (jj < ri))
            parts.append(jnp.sum(less.astype(jnp.float32), axis=1))
        rank_ref[pl.ds(b0, 128)] = jnp.concatenate(parts)
        return 0

    lax.fori_loop(0, _NB, rank_block, 0)

    kio = lax.broadcasted_iota(jnp.int32, (_NSEL, 1), 0).astype(jnp.float32)

    def sel_chunk(ch, carry):
        ar, ac = carry
        c0 = pl.multiple_of(ch * 128, 128)
        rk = rank_ref[pl.ds(c0, 128)]
        eq = (rk[None, :] == kio).astype(jnp.int32)
        lr = lr_ref[pl.ds(c0, 128)]
        lc = lc_ref[pl.ds(c0, 128)]
        ar = ar + jnp.sum(eq * lr[None, :], axis=1)
        ac = ac + jnp.sum(eq * lc[None, :], axis=1)
        return ar, ac

    zero = jnp.zeros((_NSEL,), jnp.int32)
    ar, ac = lax.fori_loop(0, _NB, sel_chunk, (zero, zero))
    ids_ref[:, 0:1] = ar[:, None]
    ids_ref[:, 1:2] = ac[:, None]

    cell0 = (ar // 8) * 64 + (ac // 8)
    cell1 = (ar // 16) * 32 + (ac // 16)

    hw0 = lax.broadcasted_iota(jnp.int32, (_NSEL, 4096), 1)
    oh0 = (cell0[:, None] == hw0).astype(jnp.float32)
    hw1 = lax.broadcasted_iota(jnp.int32, (_NSEL, 1024), 1)
    oh1 = (cell1[:, None] == hw1).astype(jnp.float32)

    dn = (((1,), (1,)), ((), ()))
    g0 = lax.dot_general(oh0, f0_ref[...], dn,
                         preferred_element_type=jnp.float32)
    g1 = lax.dot_general(oh1, f1_ref[...], dn,
                         preferred_element_type=jnp.float32)

    hp = jax.lax.Precision.HIGHEST
    h0 = jnp.maximum(
        jnp.dot(g0, w0a_ref[...], precision=hp,
                preferred_element_type=jnp.float32) + b0a_ref[...][None, :],
        0.0)
    y0 = jnp.dot(h0, w0b_ref[...], precision=hp,
                 preferred_element_type=jnp.float32) + b0b_ref[...][None, :]
    n0 = jnp.maximum(jnp.sqrt(jnp.sum(y0 * y0, axis=1, keepdims=True)),
                     1e-12)
    out0_ref[...] = y0 / n0

    h1 = jnp.maximum(
        jnp.dot(g1, w1a_ref[...], precision=hp,
                preferred_element_type=jnp.float32) + b1a_ref[...][None, :],
        0.0)
    y1 = jnp.dot(h1, w1b_ref[...], precision=hp,
                 preferred_element_type=jnp.float32) + b1b_ref[...][None, :]
    n1 = jnp.maximum(jnp.sqrt(jnp.sum(y1 * y1, axis=1, keepdims=True)),
                     1e-12)
    out1_ref[...] = y1 / n1


def kernel(img, feat0, feat1, sample_patch, sample_locs,
           W0a, b0a, W0b, b0b, W1a, b1a, W1b, b1b):
    img3 = img[0]
    sp = sample_patch[0]
    f0 = feat0[0].reshape(256, 64 * 64)
    f1 = feat1[0].reshape(512, 32 * 32)
    locs = sample_locs.astype(jnp.int32)
    locs_w = jnp.pad(locs.reshape(_NW, 2 * _CPW),
                     ((0, 0), (0, 20)))

    scores_w = _sc_scores(img3, locs_w, sp)
    scores = scores_w[:, :_CPW].reshape(_COUNT)
    scores_pad = jnp.concatenate(
        [scores, jnp.full((_PAD - _COUNT,), _BIG, jnp.float32)])

    lr = jnp.pad(locs[:, 0], (0, _PAD - _COUNT))
    lc = jnp.pad(locs[:, 1], (0, _PAD - _COUNT))

    out0, out1, ids = pl.pallas_call(
        _tc_body,
        in_specs=[pl.BlockSpec(memory_space=pltpu.VMEM)] * 13,
        out_specs=[pl.BlockSpec(memory_space=pltpu.VMEM)] * 3,
        scratch_shapes=[
            pltpu.VMEM((_PAD,), jnp.float32),
        ],
        out_shape=[
            jax.ShapeDtypeStruct((_NSEL, _NC), jnp.float32),
            jax.ShapeDtypeStruct((_NSEL, _NC), jnp.float32),
            jax.ShapeDtypeStruct((_NSEL, 2), jnp.int32),
        ],
    )(scores_pad, lr, lc, f0, f1,
      W0a, b0a, W0b, b0b, W1a, b1a, W1b, b1b)
    return out0, out1, ids
